# Initial kernel scaffold; baseline (speedup 1.0000x reference)
import jax
import jax.numpy as jnp
from jax.experimental import pallas as pl


def kernel(input_tensor, segment_label, token_table, segm_table):
    raise NotImplementedError("write your pallas kernel here")


# speedup vs baseline: 5.4176x; 5.4176x over previous
import functools

import jax
import jax.numpy as jnp
import numpy as np
from jax import lax
from jax.experimental import pallas as pl
from jax.experimental.pallas import tpu as pltpu
from jax.experimental.pallas import tpu_sc as plsc

B, L, V, D = 1024, 200, 100000, 128
N = B * L
NC, NS = 2, 16
NW = NC * NS
PER_W = N // NW
C = 128
NCHUNK = PER_W // C
LANES = 16


def _pos_embed_np():
    pos = np.arange(L, dtype=np.float32)[:, None]
    ids = np.arange(D)
    even = (ids % 2) == 0
    exponent = np.where(even, ids, ids - 1).astype(np.float32) / D
    angle = pos / np.power(10000.0, exponent)[None, :]
    pe = np.where(even[None, :], np.sin(angle), np.cos(angle))
    return pe.astype(np.float32)


_MESH = plsc.VectorSubcoreMesh(
    core_axis_name="c", subcore_axis_name="s", num_cores=NC, num_subcores=NS
)


@functools.partial(
    pl.kernel,
    out_type=jax.ShapeDtypeStruct((N, D), jnp.float32),
    mesh=_MESH,
    scratch_types=[
        pltpu.VMEM((C,), jnp.int32),
        pltpu.VMEM((C,), jnp.int32),
        pltpu.VMEM((C, D), jnp.float32),
        pltpu.VMEM((C, D), jnp.float32),
        pltpu.SemaphoreType.DMA,
        pltpu.SemaphoreType.DMA,
    ],
)
def _embed_kernel(tok_tab, combo_tab, tok_idx, seg_lab, out,
                  ti_v, ci_v, bt_v, bc_v, sem_t, sem_c):
    wid = lax.axis_index("s") * NC + lax.axis_index("c")
    wbase = wid * PER_W

    @pl.loop(0, NCHUNK)
    def _chunk(c):
        base = wbase + c * C
        pltpu.sync_copy(tok_idx.at[pl.ds(base, C)], ti_v)
        pltpu.sync_copy(seg_lab.at[pl.ds(base, C)], ci_v)

        @pl.loop(0, C // LANES)
        def _mkidx(j):
            sl = pl.ds(j * LANES, LANES)
            rows = base + j * LANES + lax.iota(jnp.int32, 16)
            ci_v[sl] = ci_v[sl] * L + lax.rem(rows, L)

        gt = pltpu.async_copy(tok_tab.at[ti_v], bt_v, sem_t)
        gc = pltpu.async_copy(combo_tab.at[ci_v], bc_v, sem_c)
        gt.wait()
        gc.wait()

        @pl.loop(0, C)
        def _add_row(r):
            for j in range(D // LANES):
                sl = pl.ds(j * LANES, LANES)
                bt_v[r, sl] += bc_v[r, sl]

        pltpu.sync_copy(bt_v, out.at[pl.ds(base, C)])


def kernel(input_tensor, segment_label, token_table, segm_table):
    pe = jnp.asarray(_pos_embed_np())
    combo_tab = (segm_table[:, None, :] + pe[None, :, :]).reshape(3 * L, D)
    tok_idx = input_tensor.reshape(-1).astype(jnp.int32)
    seg_lab = segment_label.reshape(-1).astype(jnp.int32)
    out = _embed_kernel(token_table, combo_tab, tok_idx, seg_lab)
    return out.reshape(B, L, D)
